# baseline (speedup 1.0000x reference)
import jax
import jax.numpy as jnp
from jax import lax
from jax.experimental import pallas as pl
from jax.experimental.pallas import tpu as pltpu

_RATIO = 3.0
_ROWS = 2368


def _main_body(conf_ref, locp_ref, loco_ref, orac_ref, rank_ref, cnt_ref,
               out_ref, acc_ref):
    i = pl.program_id(0)
    nsteps = pl.num_programs(0)
    conf = conf_ref[...]
    lab = orac_ref[...]
    rank = rank_ref[...]
    m = jnp.max(conf, axis=1, keepdims=True)
    e = jnp.exp(conf - m)
    s = jnp.sum(e, axis=1, keepdims=True)
    lse = m + jnp.log(s)
    iota = lax.broadcasted_iota(jnp.int32, conf.shape, 1)
    ll = jnp.sum(jnp.where(iota == lab, conf, 0.0), axis=1, keepdims=True)
    ce = lse - ll
    pos = lab > 0
    npos = jnp.sum(cnt_ref[...]).astype(jnp.float32)
    nneg = (_RATIO * npos).astype(jnp.int32)
    sel = jnp.logical_or(pos, rank < nneg)
    csum = jnp.sum(jnp.where(sel, ce, 0.0))
    d = locp_ref[...] - loco_ref[...]
    ad = jnp.abs(d)
    sl1 = jnp.where(ad < 1.0, 0.5 * d * d, ad - 0.5)
    lsum = jnp.sum(jnp.where(pos, sl1, 0.0))
    prev = jnp.where(i == 0, 0.0, acc_ref[0])
    tot = prev + csum + lsum
    acc_ref[0] = tot

    @pl.when(i == nsteps - 1)
    def _():
        out_ref[0, 0] = tot / npos


def _dense_loss(conf_r, locp_r, loco_r, orac_r, rank_r, cnt):
    np_rows, c = conf_r.shape
    grid = (np_rows // _ROWS,)
    return pl.pallas_call(
        _main_body,
        grid=grid,
        in_specs=[
            pl.BlockSpec((_ROWS, c), lambda i: (i, 0)),
            pl.BlockSpec((_ROWS, 4), lambda i: (i, 0)),
            pl.BlockSpec((_ROWS, 4), lambda i: (i, 0)),
            pl.BlockSpec((_ROWS, 1), lambda i: (i, 0)),
            pl.BlockSpec((_ROWS, 1), lambda i: (i, 0)),
            pl.BlockSpec((32, 16), lambda i: (0, 0)),
        ],
        out_specs=pl.BlockSpec(memory_space=pltpu.SMEM),
        out_shape=jax.ShapeDtypeStruct((1, 1), jnp.float32),
        scratch_shapes=[pltpu.SMEM((1,), jnp.float32)],
    )(conf_r, locp_r, loco_r, orac_r, rank_r, cnt)


def kernel(confidence_predictions, location_predictions, confidence_oracles,
           location_oracles):
    n, p, c = confidence_predictions.shape
    neg = (confidence_oracles <= 0).astype(jnp.int32)
    ranks = jnp.cumsum(neg, axis=1) - neg
    poscnt = jnp.sum(confidence_oracles > 0, axis=1, dtype=jnp.int32)
    cnt = jnp.pad(poscnt[:, None], ((0, 0), (0, 15)))
    conf_r = confidence_predictions.reshape(n * p, c)
    locp_r = location_predictions.reshape(n * p, 4)
    loco_r = location_oracles.reshape(n * p, 4)
    orac_r = confidence_oracles.reshape(n * p, 1)
    rank_r = ranks.reshape(n * p, 1)
    out = _dense_loss(conf_r, locp_r, loco_r, orac_r, rank_r, cnt)
    return out[0, 0]
